# baseline (device time: 20478 ns/iter reference)
import jax
import jax.numpy as jnp
from jax import lax
from jax.experimental import pallas as pl
from jax.experimental.pallas import tpu as pltpu

N_DEV = 4


def kernel(partial, resid, gamma):
    x = partial.reshape(partial.shape[-2], partial.shape[-1])
    m, n = x.shape
    half = m // 2
    quart = m // 4
    eighth = m // 8
    th32 = m // 16
    gamma2d = gamma.reshape(1, n)

    def body(x_hbm, resid_hbm, gamma_hbm, out_hbm,
             xv, rA1, rB1, rA2, rB2, fin, rg3, resid_v, gv,
             send_sems, recv_sems, csems):
        my = lax.axis_index("i")
        pa = my ^ 1
        pb = 3 - my

        kA1 = (my ^ (my >> 1)) & 1
        kA2 = my >> 1
        kB1 = my >> 1
        kB2 = my & 1

        A_keep1 = kA1 * quart
        A_send1 = (1 - kA1) * quart
        fwd2A = A_keep1 + (1 - kA2) * eighth
        own2A = A_keep1 + kA2 * eighth
        c0A = A_send1 + (1 - kA2) * eighth
        c1A = A_send1 + kA2 * eighth

        B_keep1 = half + kB1 * quart
        B_send1 = half + (1 - kB1) * quart
        fwd2B = B_keep1 + (1 - kB2) * eighth
        own2B = B_keep1 + kB2 * eighth
        c0B = B_send1 + kB2 * eighth
        c1B = B_send1 + (1 - kB2) * eighth

        def lcp(src_ref, src_start, rows, dst_ref, dst_start, sem_idx):
            return pltpu.make_async_copy(
                src_ref.at[pl.ds(src_start, rows), :],
                dst_ref.at[pl.ds(dst_start, rows), :],
                csems.at[sem_idx],
            )

        xcp = []
        for i, st in enumerate((c0A, c0B, c1A, c1B, fwd2A, fwd2B, own2A, own2B)):
            c = lcp(x_hbm, st, eighth, xv, st, i)
            c.start()
            xcp.append(c)
        rcp0 = lcp(resid_hbm, own2A, eighth, resid_v, 0, 8)
        rcp1 = lcp(resid_hbm, own2B, eighth, resid_v, eighth, 9)
        rcp0.start()
        rcp1.start()
        gcp = pltpu.make_async_copy(gamma_hbm, gv, csems.at[10])
        gcp.start()

        barrier_sem = pltpu.get_barrier_semaphore()
        for nbr in [pa, pb]:
            pl.semaphore_signal(
                barrier_sem, inc=1,
                device_id=(nbr,), device_id_type=pl.DeviceIdType.MESH,
            )
        pl.semaphore_wait(barrier_sem, 2)

        def rc(src_ref, src_start, rows, dst_ref, dst_start, peer, idx):
            return pltpu.make_async_remote_copy(
                src_ref=src_ref.at[pl.ds(src_start, rows), :],
                dst_ref=dst_ref.at[pl.ds(dst_start, rows), :],
                send_sem=send_sems.at[idx],
                recv_sem=recv_sems.at[idx],
                device_id=(peer,),
                device_id_type=pl.DeviceIdType.MESH,
            )

        xcp[0].wait()
        s1af0 = rc(xv, c0A, th32, rA1, 0, pa, 0)
        s1af1 = rc(xv, c0A + th32, th32, rA1, th32, pa, 1)
        s1af0.start()
        s1af1.start()
        xcp[1].wait()
        s1bf0 = rc(xv, c0B, th32, rB1, 0, pb, 3)
        s1bf1 = rc(xv, c0B + th32, th32, rB1, th32, pb, 4)
        s1bf0.start()
        s1bf1.start()
        xcp[2].wait()
        s1ao = rc(xv, c1A, eighth, rA1, eighth, pa, 2)
        s1ao.start()
        xcp[3].wait()
        s1bo = rc(xv, c1B, eighth, rB1, eighth, pb, 5)
        s1bo.start()

        xcp[4].wait()
        s1af0.wait_recv()
        xv[pl.ds(fwd2A, th32), :] = (
            xv[pl.ds(fwd2A, th32), :] + rA1[pl.ds(0, th32), :]
        )
        s2a0 = rc(xv, fwd2A, th32, rA2, 0, pb, 6)
        s2a0.start()

        xcp[5].wait()
        s1bf0.wait_recv()
        xv[pl.ds(fwd2B, th32), :] = (
            xv[pl.ds(fwd2B, th32), :] + rB1[pl.ds(0, th32), :]
        )
        s2b0 = rc(xv, fwd2B, th32, rB2, 0, pa, 8)
        s2b0.start()

        s1af1.wait_recv()
        xv[pl.ds(fwd2A + th32, th32), :] = (
            xv[pl.ds(fwd2A + th32, th32), :] + rA1[pl.ds(th32, th32), :]
        )
        s2a1 = rc(xv, fwd2A + th32, th32, rA2, th32, pb, 7)
        s2a1.start()

        s1bf1.wait_recv()
        xv[pl.ds(fwd2B + th32, th32), :] = (
            xv[pl.ds(fwd2B + th32, th32), :] + rB1[pl.ds(th32, th32), :]
        )
        s2b1 = rc(xv, fwd2B + th32, th32, rB2, th32, pa, 9)
        s2b1.start()

        xcp[6].wait()
        s1ao.wait_recv()
        xv[pl.ds(own2A, eighth), :] = (
            xv[pl.ds(own2A, eighth), :] + rA1[pl.ds(eighth, eighth), :]
        )
        xcp[7].wait()
        s1bo.wait_recv()
        xv[pl.ds(own2B, eighth), :] = (
            xv[pl.ds(own2B, eighth), :] + rB1[pl.ds(eighth, eighth), :]
        )

        rcp0.wait()
        rcp1.wait()
        gcp.wait()
        g = gv[0, :][None, :]

        out_cps = []

        def ln_and_gather(rdma_in, start, rbuf, roff, foff, p3, p4,
                          i3, i4, ic):
            rdma_in.wait_recv()
            y = (
                xv[pl.ds(start, th32), :]
                + rbuf[pl.ds(roff, th32), :]
                + resid_v[pl.ds(foff, th32), :]
            )
            rms = jnp.sqrt(jnp.mean(y * y, axis=-1, keepdims=True) + 1e-6)
            fin[pl.ds(foff, th32), :] = y / rms * g
            g3 = rc(fin, foff, th32, rg3, foff, p3, i3)
            g4 = rc(fin, foff, th32, out_hbm, start, p4, i4)
            g3.start()
            g4.start()
            oc = lcp(fin, foff, th32, out_hbm, start, ic)
            oc.start()
            out_cps.append(oc)
            return g3, g4

        g3a0, g4aa0 = ln_and_gather(s2a0, own2A, rA2, 0, 0, pb, pa, 10, 14, 11)
        g3b0, g4ab0 = ln_and_gather(
            s2b0, own2B, rB2, 0, 2 * th32, pa, pb, 12, 16, 12
        )
        g3a1, g4aa1 = ln_and_gather(
            s2a1, own2A + th32, rA2, th32, th32, pb, pa, 11, 15, 13
        )
        g3b1, g4ab1 = ln_and_gather(
            s2b1, own2B + th32, rB2, th32, 3 * th32, pa, pb, 13, 17, 14
        )

        def fwd_out(rdma_in, foff, gstart, peer, idx, ic):
            rdma_in.wait_recv()
            f = rc(rg3, foff, th32, out_hbm, gstart, peer, idx)
            f.start()
            oc = lcp(rg3, foff, th32, out_hbm, gstart, ic)
            oc.start()
            out_cps.append(oc)
            return f

        g4ba0 = fwd_out(g3a0, 0, fwd2A, pa, 18, 15)
        g4bb0 = fwd_out(g3b0, 2 * th32, fwd2B, pb, 20, 16)
        g4ba1 = fwd_out(g3a1, th32, fwd2A + th32, pa, 19, 17)
        g4bb1 = fwd_out(g3b1, 3 * th32, fwd2B + th32, pb, 21, 18)

        for r in (g4aa0, g4aa1, g4ab0, g4ab1, g4ba0, g4ba1, g4bb0, g4bb1):
            r.wait_recv()

        for c in out_cps:
            c.wait()
        for r in (s1af0, s1af1, s1ao, s1bf0, s1bf1, s1bo,
                  s2a0, s2a1, s2b0, s2b1,
                  g3a0, g3a1, g3b0, g3b1, g4aa0, g4aa1, g4ab0, g4ab1,
                  g4ba0, g4ba1, g4bb0, g4bb1):
            r.wait_send()

    return pl.pallas_call(
        body,
        out_shape=jax.ShapeDtypeStruct((m, n), jnp.float32),
        in_specs=[
            pl.BlockSpec(memory_space=pl.ANY),
            pl.BlockSpec(memory_space=pl.ANY),
            pl.BlockSpec(memory_space=pl.ANY),
        ],
        out_specs=pl.BlockSpec(memory_space=pl.ANY),
        scratch_shapes=[
            pltpu.VMEM((m, n), jnp.float32),
            pltpu.VMEM((quart, n), jnp.float32),
            pltpu.VMEM((quart, n), jnp.float32),
            pltpu.VMEM((eighth, n), jnp.float32),
            pltpu.VMEM((eighth, n), jnp.float32),
            pltpu.VMEM((quart, n), jnp.float32),
            pltpu.VMEM((quart, n), jnp.float32),
            pltpu.VMEM((quart, n), jnp.float32),
            pltpu.VMEM((1, n), jnp.float32),
            pltpu.SemaphoreType.DMA((22,)),
            pltpu.SemaphoreType.DMA((22,)),
            pltpu.SemaphoreType.DMA((19,)),
        ],
        compiler_params=pltpu.CompilerParams(collective_id=0),
    )(x, resid, gamma2d)


# device time: 17931 ns/iter; 1.1420x vs baseline; 1.1420x over previous
import jax
import jax.numpy as jnp
from jax import lax
from jax.experimental import pallas as pl
from jax.experimental.pallas import tpu as pltpu

N_DEV = 4
NCHUNK = 4


def kernel(partial, resid, gamma):
    x = partial.reshape(partial.shape[-2], partial.shape[-1])
    m, n = x.shape
    half = m // 2
    quart = m // 4
    ch = quart // NCHUNK
    gamma2d = gamma.reshape(1, n)

    def body(x_ref, resid_hbm, gamma_ref, out_ref,
             resid_v, rA1, rB1, rA2, rB2, send_sems, recv_sems, copy_sem):
        my = lax.axis_index("i")
        pa = my ^ 1
        pb = 3 - my

        kA1 = (my ^ (my >> 1)) & 1
        kB1 = my >> 1

        A_keep = kA1 * quart
        A_send = (1 - kA1) * quart
        B_keep = half + kB1 * quart
        B_send = half + (1 - kB1) * quart

        cp = pltpu.make_async_copy(resid_hbm, resid_v, copy_sem)
        cp.start()

        barrier_sem = pltpu.get_barrier_semaphore()
        for nbr in [pa, pb]:
            pl.semaphore_signal(
                barrier_sem, inc=1,
                device_id=(nbr,), device_id_type=pl.DeviceIdType.MESH,
            )
        pl.semaphore_wait(barrier_sem, 2)

        def rc(src_ref, src_start, rows, dst_ref, dst_start, peer, idx):
            return pltpu.make_async_remote_copy(
                src_ref=src_ref.at[pl.ds(src_start, rows), :],
                dst_ref=dst_ref.at[pl.ds(dst_start, rows), :],
                send_sem=send_sems.at[idx],
                recv_sem=recv_sems.at[idx],
                device_id=(peer,),
                device_id_type=pl.DeviceIdType.MESH,
            )

        s1a = [rc(x_ref, A_send + k * ch, ch, rA1, k * ch, pa, k)
               for k in range(NCHUNK)]
        s1b = [rc(x_ref, B_send + k * ch, ch, rB1, k * ch, pb, NCHUNK + k)
               for k in range(NCHUNK)]
        for k in range(NCHUNK):
            s1a[k].start()
            s1b[k].start()

        s2a = []
        s2b = []
        for k in range(NCHUNK):
            s1a[k].wait_recv()
            out_ref[pl.ds(A_keep + k * ch, ch), :] = (
                x_ref[pl.ds(A_keep + k * ch, ch), :] + rA1[pl.ds(k * ch, ch), :]
            )
            r = rc(out_ref, A_keep + k * ch, ch, rA2, k * ch, pb, 2 * NCHUNK + k)
            r.start()
            s2a.append(r)

            s1b[k].wait_recv()
            out_ref[pl.ds(B_keep + k * ch, ch), :] = (
                x_ref[pl.ds(B_keep + k * ch, ch), :] + rB1[pl.ds(k * ch, ch), :]
            )
            r = rc(out_ref, B_keep + k * ch, ch, rB2, k * ch, pa, 3 * NCHUNK + k)
            r.start()
            s2b.append(r)

        cp.wait()
        g = gamma_ref[0, :][None, :]

        g3a = []
        g3b = []
        for k in range(NCHUNK):
            s2a[k].wait_recv()
            start = A_keep + k * ch
            y = (
                out_ref[pl.ds(start, ch), :]
                + rA2[pl.ds(k * ch, ch), :]
                + resid_v[pl.ds(start, ch), :]
            )
            rms = jnp.sqrt(jnp.mean(y * y, axis=-1, keepdims=True) + 1e-6)
            out_ref[pl.ds(start, ch), :] = y / rms * g
            r = rc(out_ref, start, ch, out_ref, start, pa, 4 * NCHUNK + k)
            r.start()
            g3a.append(r)

            s2b[k].wait_recv()
            start = B_keep + k * ch
            y = (
                out_ref[pl.ds(start, ch), :]
                + rB2[pl.ds(k * ch, ch), :]
                + resid_v[pl.ds(start, ch), :]
            )
            rms = jnp.sqrt(jnp.mean(y * y, axis=-1, keepdims=True) + 1e-6)
            out_ref[pl.ds(start, ch), :] = y / rms * g
            r = rc(out_ref, start, ch, out_ref, start, pb, 5 * NCHUNK + k)
            r.start()
            g3b.append(r)

        for r in g3a + g3b:
            r.wait_recv()
        for r in s1a + s1b + s2a + s2b + g3a + g3b:
            r.wait_send()

    return pl.pallas_call(
        body,
        out_shape=jax.ShapeDtypeStruct((m, n), jnp.float32),
        in_specs=[
            pl.BlockSpec(memory_space=pltpu.VMEM),
            pl.BlockSpec(memory_space=pl.ANY),
            pl.BlockSpec(memory_space=pltpu.VMEM),
        ],
        out_specs=pl.BlockSpec(memory_space=pltpu.VMEM),
        scratch_shapes=[
            pltpu.VMEM((m, n), jnp.float32),
            pltpu.VMEM((quart, n), jnp.float32),
            pltpu.VMEM((quart, n), jnp.float32),
            pltpu.VMEM((quart, n), jnp.float32),
            pltpu.VMEM((quart, n), jnp.float32),
            pltpu.SemaphoreType.DMA((24,)),
            pltpu.SemaphoreType.DMA((24,)),
            pltpu.SemaphoreType.DMA,
        ],
        compiler_params=pltpu.CompilerParams(collective_id=0),
    )(x, resid, gamma2d)
